# Initial kernel scaffold; baseline (speedup 1.0000x reference)
import jax
import jax.numpy as jnp
from jax.experimental import pallas as pl


def kernel(c0, src, dst, R_src, R_dst, poly_coeffs):
    raise NotImplementedError("write your pallas kernel here")


# speedup vs baseline: 202.1079x; 202.1079x over previous
import functools

import jax
import jax.numpy as jnp
from jax import lax
from jax.experimental import pallas as pl
from jax.experimental.pallas import tpu as pltpu
from jax.experimental.pallas import tpu_sc as plsc

_N = 100000
_E = 1600000
_BD = 8
_RF = 16
_NC = 2
_NS = 16
_NW = _NC * _NS
_EPT = _E // _NW
_C = 400
_NCHUNK = _EPT // _C
_G = _C // 16
_SUB = 50
_NSUB = _C // _SUB
_NSTAGE = 10
_RPT = _N // _NSTAGE
_SC = 1000


def _sheaf_step_body(p_hbm, srcr, dstr, rs_hbm, rd_hbm, zero_hbm, out_hbm,
                     acc, idx_s, idx_d, rs_v, rd_v, ps_v, pd_v,
                     cs_v, cd_v, stage, sem):
    c = lax.axis_index("c")
    s = lax.axis_index("s")
    w = c * _NS + s

    r0 = s * _RPT

    @pl.when(s < _NSTAGE)
    def _():
        def zero_body(j, carry):
            rr = r0 + j * _SC
            pltpu.sync_copy(zero_hbm.at[pl.ds(rr, _SC)], stage)
            pltpu.sync_copy(stage, acc.at[pl.ds(rr, _SC)])
            return carry

        lax.fori_loop(0, _RPT // _SC, zero_body, 0)

    plsc.subcore_barrier()

    iota = lax.iota(jnp.int32, 16)
    e0 = w * _EPT
    ir0 = w * (_EPT // _SUB)

    def chunk_body(i, carry):
        erow = e0 + i * _C
        irow = ir0 + i * _NSUB
        pltpu.sync_copy(srcr.at[pl.ds(irow, _NSUB)], idx_s)
        pltpu.sync_copy(dstr.at[pl.ds(irow, _NSUB)], idx_d)
        pltpu.sync_copy(rs_hbm.at[pl.ds(erow, _C)], rs_v)
        pltpu.sync_copy(rd_hbm.at[pl.ds(erow, _C)], rd_v)
        cps = [pltpu.async_copy(p_hbm.at[idx_s.at[j]],
                                ps_v.at[pl.ds(j * _SUB, _SUB)], sem)
               for j in range(_NSUB)]
        cpd = [pltpu.async_copy(p_hbm.at[idx_d.at[j]],
                                pd_v.at[pl.ds(j * _SUB, _SUB)], sem)
               for j in range(_NSUB)]
        for cp in cps + cpd:
            cp.wait()

        def group_body(g, carry2):
            rows = g * 16 + iota
            cols = [jnp.full((16,), j, jnp.int32) for j in range(16)]
            Rs = [plsc.load_gather(rs_v, [rows, cols[j]]) for j in range(16)]
            Rd = [plsc.load_gather(rd_v, [rows, cols[j]]) for j in range(16)]
            Ps = [plsc.load_gather(ps_v, [rows, cols[j]]) for j in range(8)]
            Pd = [plsc.load_gather(pd_v, [rows, cols[j]]) for j in range(8)]
            for b in range(2):
                r = []
                for a in range(4):
                    t = Rs[a * 4] * Ps[b * 4]
                    u = Rd[a * 4] * Pd[b * 4]
                    for d in range(1, 4):
                        t = t + Rs[a * 4 + d] * Ps[b * 4 + d]
                        u = u + Rd[a * 4 + d] * Pd[b * 4 + d]
                    r.append(t - u)
                for d in range(4):
                    cs = Rs[d] * r[0]
                    cd = Rd[d] * r[0]
                    for a in range(1, 4):
                        cs = cs + Rs[a * 4 + d] * r[a]
                        cd = cd + Rd[a * 4 + d] * r[a]
                    plsc.store_scatter(cs_v, [rows, cols[b * 4 + d]], cs)
                    plsc.store_scatter(cd_v, [rows, cols[b * 4 + d]], -cd)
            return carry2

        lax.fori_loop(0, _G, group_body, 0)

        for j in range(_NSUB):
            pltpu.sync_copy(cs_v.at[pl.ds(j * _SUB, _SUB)],
                            acc.at[idx_s.at[j]], add=True)
            pltpu.sync_copy(cd_v.at[pl.ds(j * _SUB, _SUB)],
                            acc.at[idx_d.at[j]], add=True)
        return carry

    lax.fori_loop(0, _NCHUNK, chunk_body, 0)

    plsc.subcore_barrier()

    @pl.when(s < _NSTAGE)
    def _():
        def wb_body(j, carry):
            rr = r0 + j * _SC
            pltpu.sync_copy(acc.at[pl.ds(rr, _SC)], stage)
            pltpu.sync_copy(stage, out_hbm.at[pl.ds(c * _N + rr, _SC)])
            return carry

        lax.fori_loop(0, _RPT // _SC, wb_body, 0)


_sheaf_step = functools.partial(
    pl.kernel,
    out_type=jax.ShapeDtypeStruct((_NC * _N, _BD), jnp.float32),
    mesh=plsc.VectorSubcoreMesh(core_axis_name="c", subcore_axis_name="s"),
    scratch_types=[
        pltpu.VMEM_SHARED((_N, _BD), jnp.float32),
        pltpu.VMEM((_NSUB, _SUB), jnp.int32),
        pltpu.VMEM((_NSUB, _SUB), jnp.int32),
        pltpu.VMEM((_C, _RF), jnp.float32),
        pltpu.VMEM((_C, _RF), jnp.float32),
        pltpu.VMEM((_C, _BD), jnp.float32),
        pltpu.VMEM((_C, _BD), jnp.float32),
        pltpu.VMEM((_C, _BD), jnp.float32),
        pltpu.VMEM((_C, _BD), jnp.float32),
        pltpu.VMEM((_SC, _BD), jnp.float32),
        pltpu.SemaphoreType.DMA,
    ],
    compiler_params=pltpu.CompilerParams(
        needs_layout_passes=False, use_tc_tiling_on_sc=False),
)(_sheaf_step_body)


def kernel(c0, src, dst, R_src, R_dst, poly_coeffs):
    B, N, D = c0.shape
    E = src.shape[0]
    p = jnp.transpose(c0, (1, 0, 2)).reshape(N, B * D)
    srcr = src.astype(jnp.int32).reshape(E // _SUB, _SUB)
    dstr = dst.astype(jnp.int32).reshape(E // _SUB, _SUB)
    rs = R_src.reshape(E, _RF)
    rd = R_dst.reshape(E, _RF)
    zero = jnp.zeros((N, B * D), jnp.float32)

    out = poly_coeffs[0] * p
    v = p
    for k in range(1, 4):
        parts = _sheaf_step(v, srcr, dstr, rs, rd, zero)
        v = parts[:N] + parts[N:]
        out = out + poly_coeffs[k] * v
    return out.reshape(N, B, D).transpose(1, 0, 2)


# speedup vs baseline: 324.7614x; 1.6069x over previous
import functools

import jax
import jax.numpy as jnp
from jax import lax
from jax.experimental import pallas as pl
from jax.experimental.pallas import tpu as pltpu
from jax.experimental.pallas import tpu_sc as plsc

_N = 100000
_E = 1600000
_BD = 8
_RF = 16
_NC = 2
_NS = 16
_NW = _NC * _NS
_EPT = _E // _NW
_C = 400
_NCHUNK = _EPT // _C
_PAIRS = (_NCHUNK - 1) // 2
_G = _C // 16
_SUB = 50
_NSUB = _C // _SUB
_IPT = _EPT // _SUB
_NSTAGE = 10
_RPT = _N // _NSTAGE
_SC = 1000


def _compute_chunk(rs_v, rd_v, ps_v, pd_v, cs_v, cd_v, iota):
    def group_body(g, carry):
        rows = g * 16 + iota
        cols = [jnp.full((16,), j, jnp.int32) for j in range(16)]
        Rs = [plsc.load_gather(rs_v, [rows, cols[j]]) for j in range(16)]
        Rd = [plsc.load_gather(rd_v, [rows, cols[j]]) for j in range(16)]
        Ps = [plsc.load_gather(ps_v, [rows, cols[j]]) for j in range(8)]
        Pd = [plsc.load_gather(pd_v, [rows, cols[j]]) for j in range(8)]
        for b in range(2):
            r = []
            for a in range(4):
                t = Rs[a * 4] * Ps[b * 4]
                u = Rd[a * 4] * Pd[b * 4]
                for d in range(1, 4):
                    t = t + Rs[a * 4 + d] * Ps[b * 4 + d]
                    u = u + Rd[a * 4 + d] * Pd[b * 4 + d]
                r.append(t - u)
            for d in range(4):
                cs = Rs[d] * r[0]
                cd = Rd[d] * r[0]
                for a in range(1, 4):
                    cs = cs + Rs[a * 4 + d] * r[a]
                    cd = cd + Rd[a * 4 + d] * r[a]
                plsc.store_scatter(cs_v, [rows, cols[b * 4 + d]], cs)
                plsc.store_scatter(cd_v, [rows, cols[b * 4 + d]], -cd)
        return carry

    lax.fori_loop(0, _G, group_body, 0)


def _sheaf_step_body(p_hbm, srcr, dstr, rs_hbm, rd_hbm, zero_hbm, out_hbm,
                     acc,
                     igA, igB, isA, isB,
                     rsA, rdA, rsB, rdB,
                     psA, pdA, psB, pdB,
                     csA, cdA, csB, cdB,
                     stage,
                     s_igA, s_igB, s_isA, s_isB,
                     s_inA, s_inB, s_scA, s_scB):
    c = lax.axis_index("c")
    s = lax.axis_index("s")
    w = c * _NS + s

    r0 = s * _RPT

    @pl.when(s < _NSTAGE)
    def _():
        def zero_body(j, carry):
            rr = r0 + j * _SC
            pltpu.sync_copy(zero_hbm.at[pl.ds(rr, _SC)], stage)
            pltpu.sync_copy(stage, acc.at[pl.ds(rr, _SC)])
            return carry

        lax.fori_loop(0, _RPT // _SC, zero_body, 0)

    plsc.subcore_barrier()

    iota = lax.iota(jnp.int32, 16)
    e0 = w * _EPT
    ir0 = w * _IPT
    last = _NCHUNK - 1

    bufs = {
        0: (igA, isA, rsA, rdA, psA, pdA, csA, cdA, s_igA, s_isA, s_inA, s_scA),
        1: (igB, isB, rsB, rdB, psB, pdB, csB, cdB, s_igB, s_isB, s_inB, s_scB),
    }

    def issue_inputs(q, ph):
        ig, _, rs_v, rd_v, ps_v, pd_v, _, _, _, _, s_in, _ = bufs[ph]
        erow = e0 + q * _C
        pltpu.async_copy(rs_hbm.at[pl.ds(erow, _C)], rs_v, s_in)
        pltpu.async_copy(rd_hbm.at[pl.ds(erow, _C)], rd_v, s_in)
        for j in range(_NSUB):
            pltpu.async_copy(p_hbm.at[ig.at[j]],
                             ps_v.at[pl.ds(j * _SUB, _SUB)], s_in)
            pltpu.async_copy(p_hbm.at[ig.at[_NSUB + j]],
                             pd_v.at[pl.ds(j * _SUB, _SUB)], s_in)

    def wait_inputs(ph):
        ig, _, rs_v, rd_v, ps_v, pd_v, _, _, _, _, s_in, _ = bufs[ph]
        pltpu.make_async_copy(rs_hbm.at[pl.ds(0, _C)], rs_v, s_in).wait()
        pltpu.make_async_copy(rd_hbm.at[pl.ds(0, _C)], rd_v, s_in).wait()
        for j in range(_NSUB):
            pltpu.make_async_copy(p_hbm.at[ig.at[j]],
                                  ps_v.at[pl.ds(j * _SUB, _SUB)], s_in).wait()
            pltpu.make_async_copy(p_hbm.at[ig.at[_NSUB + j]],
                                  pd_v.at[pl.ds(j * _SUB, _SUB)], s_in).wait()

    def issue_scatters(ph):
        _, isx, _, _, _, _, cs_v, cd_v, _, _, _, s_sc = bufs[ph]
        for j in range(_NSUB):
            pltpu.async_copy(cs_v.at[pl.ds(j * _SUB, _SUB)],
                             acc.at[isx.at[j]], s_sc, add=True)
            pltpu.async_copy(cd_v.at[pl.ds(j * _SUB, _SUB)],
                             acc.at[isx.at[_NSUB + j]], s_sc, add=True)

    def wait_scatters(ph):
        _, isx, _, _, _, _, cs_v, cd_v, _, _, _, s_sc = bufs[ph]
        for j in range(_NSUB):
            pltpu.make_async_copy(cs_v.at[pl.ds(j * _SUB, _SUB)],
                                  acc.at[isx.at[j]], s_sc).wait()
            pltpu.make_async_copy(cd_v.at[pl.ds(j * _SUB, _SUB)],
                                  acc.at[isx.at[_NSUB + j]], s_sc).wait()

    def issue_idx(q, ph, which):
        ig, isx, _, _, _, _, _, _, s_ig, s_is, _, _ = bufs[ph]
        ref = ig if which == 0 else isx
        sem = s_ig if which == 0 else s_is
        irow = ir0 + q * _NSUB
        pltpu.async_copy(srcr.at[pl.ds(irow, _NSUB)],
                         ref.at[pl.ds(0, _NSUB)], sem)
        pltpu.async_copy(dstr.at[pl.ds(irow, _NSUB)],
                         ref.at[pl.ds(_NSUB, _NSUB)], sem)

    def wait_idx(ph, which):
        ig, isx, _, _, _, _, _, _, s_ig, s_is, _, _ = bufs[ph]
        ref = ig if which == 0 else isx
        sem = s_ig if which == 0 else s_is
        pltpu.make_async_copy(srcr.at[pl.ds(0, _NSUB)],
                              ref.at[pl.ds(0, _NSUB)], sem).wait()
        pltpu.make_async_copy(dstr.at[pl.ds(0, _NSUB)],
                              ref.at[pl.ds(_NSUB, _NSUB)], sem).wait()

    def compute(ph):
        _, _, rs_v, rd_v, ps_v, pd_v, cs_v, cd_v, _, _, _, _ = bufs[ph]
        _compute_chunk(rs_v, rd_v, ps_v, pd_v, cs_v, cd_v, iota)

    def phase(kk, q, ph):
        @pl.when(kk > 0)
        def _():
            wait_scatters(ph)
        issue_idx(q, ph, 1)
        wait_inputs(ph)
        issue_idx(jnp.minimum(q + 2, last), ph, 0)
        compute(ph)
        wait_idx(ph, 1)
        issue_scatters(ph)
        wait_idx(ph, 0)
        issue_inputs(jnp.minimum(q + 2, last), ph)

    issue_idx(0, 0, 0)
    wait_idx(0, 0)
    issue_inputs(0, 0)
    issue_idx(1, 1, 0)
    wait_idx(1, 0)
    issue_inputs(1, 1)

    def pair_body(kk, carry):
        phase(kk, 2 * kk, 0)
        phase(kk, 2 * kk + 1, 1)
        return carry

    lax.fori_loop(0, _PAIRS, pair_body, 0)

    q = last
    wait_scatters(0)
    issue_idx(q, 0, 1)
    wait_inputs(0)
    compute(0)
    wait_idx(0, 1)
    issue_scatters(0)
    wait_inputs(1)
    wait_scatters(0)
    wait_scatters(1)

    plsc.subcore_barrier()

    @pl.when(s < _NSTAGE)
    def _():
        def wb_body(j, carry):
            rr = r0 + j * _SC
            pltpu.sync_copy(acc.at[pl.ds(rr, _SC)], stage)
            pltpu.sync_copy(stage, out_hbm.at[pl.ds(c * _N + rr, _SC)])
            return carry

        lax.fori_loop(0, _RPT // _SC, wb_body, 0)


_sheaf_step = functools.partial(
    pl.kernel,
    out_type=jax.ShapeDtypeStruct((_NC * _N, _BD), jnp.float32),
    mesh=plsc.VectorSubcoreMesh(core_axis_name="c", subcore_axis_name="s"),
    scratch_types=[
        pltpu.VMEM_SHARED((_N, _BD), jnp.float32),
        pltpu.VMEM((2 * _NSUB, _SUB), jnp.int32),
        pltpu.VMEM((2 * _NSUB, _SUB), jnp.int32),
        pltpu.VMEM((2 * _NSUB, _SUB), jnp.int32),
        pltpu.VMEM((2 * _NSUB, _SUB), jnp.int32),
        pltpu.VMEM((_C, _RF), jnp.float32),
        pltpu.VMEM((_C, _RF), jnp.float32),
        pltpu.VMEM((_C, _RF), jnp.float32),
        pltpu.VMEM((_C, _RF), jnp.float32),
        pltpu.VMEM((_C, _BD), jnp.float32),
        pltpu.VMEM((_C, _BD), jnp.float32),
        pltpu.VMEM((_C, _BD), jnp.float32),
        pltpu.VMEM((_C, _BD), jnp.float32),
        pltpu.VMEM((_C, _BD), jnp.float32),
        pltpu.VMEM((_C, _BD), jnp.float32),
        pltpu.VMEM((_C, _BD), jnp.float32),
        pltpu.VMEM((_C, _BD), jnp.float32),
        pltpu.VMEM((_SC, _BD), jnp.float32),
        pltpu.SemaphoreType.DMA,
        pltpu.SemaphoreType.DMA,
        pltpu.SemaphoreType.DMA,
        pltpu.SemaphoreType.DMA,
        pltpu.SemaphoreType.DMA,
        pltpu.SemaphoreType.DMA,
        pltpu.SemaphoreType.DMA,
        pltpu.SemaphoreType.DMA,
    ],
    compiler_params=pltpu.CompilerParams(
        needs_layout_passes=False, use_tc_tiling_on_sc=False),
)(_sheaf_step_body)


def kernel(c0, src, dst, R_src, R_dst, poly_coeffs):
    B, N, D = c0.shape
    E = src.shape[0]
    p = jnp.transpose(c0, (1, 0, 2)).reshape(N, B * D)
    srcr = src.astype(jnp.int32).reshape(E // _SUB, _SUB)
    dstr = dst.astype(jnp.int32).reshape(E // _SUB, _SUB)
    rs = R_src.reshape(E, _RF)
    rd = R_dst.reshape(E, _RF)
    zero = jnp.zeros((N, B * D), jnp.float32)

    out = poly_coeffs[0] * p
    v = p
    for k in range(1, 4):
        parts = _sheaf_step(v, srcr, dstr, rs, rd, zero)
        v = parts[:N] + parts[N:]
        out = out + poly_coeffs[k] * v
    return out.reshape(N, B, D).transpose(1, 0, 2)
